# baseline (speedup 1.0000x reference)
import functools

import jax
import jax.numpy as jnp
from jax.experimental import pallas as pl

C = 768
D = 64
NQ = 8
KCB = 1024

_HI = jax.lax.Precision.HIGHEST
_DEF = jax.lax.Precision.DEFAULT


def _elu(x):
    return jnp.where(x > 0, x, jnp.exp(jnp.minimum(x, 0.0)) - 1.0)


def _mm(x, w, b, prec):
    y = jax.lax.dot_general(x, w, (((1,), (0,)), ((), ())),
                            precision=prec,
                            preferred_element_type=jnp.float32)
    return y + b


def _lin_p(p, x, prec):
    return _mm(x, p[0], p[1], prec)


def _ru_p(p, x, prec):
    return x + _lin_p(p[1], _elu(_lin_p(p[0], x, prec)), prec)


def _db_p(p, x, prec):
    x = _elu(_lin_p(p[0], x, prec))
    x = _elu(_ru_p(p[1], x, prec))
    x = _elu(_ru_p(p[2], x, prec))
    return _ru_p(p[3], x, prec)


def _dec_apply(p, x, prec):
    x = _elu(_lin_p(p[0], x, prec))
    x = _elu(_db_p(p[1], x, prec))
    x = _elu(_db_p(p[2], x, prec))
    return _lin_p(p[3], x, prec)


def _lin_x(p, x):
    return x @ p[0] + p[1]


def _ru_x(p, x):
    return x + _lin_x(p[1], jax.nn.elu(_lin_x(p[0], x)))


def _eb_x(p, x):
    x = _ru_x(p[0], x); x = jax.nn.elu(x)
    x = _ru_x(p[1], x); x = jax.nn.elu(x)
    x = _ru_x(p[2], x); x = jax.nn.elu(x)
    return _lin_x(p[3], x)


def _encoder_x(p, x):
    x = _lin_x(p[0], x); x = jax.nn.elu(x)
    x = _eb_x(p[1], x); x = jax.nn.elu(x)
    x = _eb_x(p[2], x); x = jax.nn.elu(x)
    x = _eb_x(p[3], x); x = jax.nn.elu(x)
    x = _eb_x(p[4], x); x = jax.nn.elu(x)
    return _lin_x(p[5], x)


def _rvq_dec_kernel(dec_tree, n_w, nblk, n_tokens,
                    e_ref, *refs):
    cbt_ref = refs[0]
    cbs_refs = refs[1:4]
    cbsq_ref = refs[4]
    w_refs = refs[5:5 + n_w]
    o_ref, idx_ref, loss_ref = refs[5 + n_w:]

    cbt = cbt_ref[...]
    cb_sq = cbsq_ref[...]
    cb1, cb2, cb3 = (r[...] for r in cbs_refs)

    e = e_ref[...]
    bsz = e.shape[0]
    iota = jax.lax.broadcasted_iota(jnp.int32, (bsz, KCB), 1)
    dn = (((1,), (0,)), ((), ()))

    residual = e
    qout = jnp.zeros_like(e)
    idx_cols = []
    loss_cols = []
    for _ in range(NQ):
        rsq = jnp.sum(residual * residual, axis=1, keepdims=True)
        mm = jax.lax.dot_general(residual, cbt, dn, precision=_DEF,
                                 preferred_element_type=jnp.float32)
        d = rsq - 2.0 * mm + cb_sq
        dmin = jnp.min(d, axis=1, keepdims=True)
        idxv = jnp.min(jnp.where(d == dmin, iota, jnp.int32(KCB)),
                       axis=1, keepdims=True)
        oh = (iota == idxv).astype(jnp.bfloat16)
        q12 = (jax.lax.dot_general(oh, cb1, dn, preferred_element_type=jnp.float32)
               + jax.lax.dot_general(oh, cb2, dn, preferred_element_type=jnp.float32))
        quant = q12 + jax.lax.dot_general(oh, cb3, dn,
                                          preferred_element_type=jnp.float32)
        diff = quant - residual
        loss_cols.append(jnp.sum(diff * diff).reshape(1, 1))
        qout = qout + (residual + diff)
        residual = residual - quant
        idx_cols.append(idxv)

    idx_ref[...] = jnp.concatenate(idx_cols, axis=1)

    i = pl.program_id(0)

    @pl.when(i == 0)
    def _zero():
        loss_ref[...] = jnp.zeros_like(loss_ref)

    loss_ref[...] += jnp.concatenate(loss_cols, axis=1)

    @pl.when(i == nblk - 1)
    def _scale():
        loss_ref[...] *= jnp.float32(1.0 / (n_tokens * D))

    dec = jax.tree.unflatten(dec_tree, [r[...] for r in w_refs])
    o_ref[...] = _dec_apply(dec, qout, _HI)


def _prep_leaves(params):
    leaves, tree = jax.tree.flatten(params)
    leaves = [l if l.ndim == 2 else l.reshape(1, -1) for l in leaves]
    return leaves, tree


def _const_spec(a):
    return pl.BlockSpec(a.shape, lambda i: (0,) * a.ndim)


def _split3(v):
    m = jnp.uint32(0xFFFF0000)
    u = jax.lax.bitcast_convert_type(v, jnp.uint32)
    v1f = jax.lax.bitcast_convert_type(u & m, jnp.float32)
    r1 = v - v1f
    u1 = jax.lax.bitcast_convert_type(r1, jnp.uint32)
    v2f = jax.lax.bitcast_convert_type(u1 & m, jnp.float32)
    r2 = r1 - v2f
    return (v1f.astype(jnp.bfloat16), v2f.astype(jnp.bfloat16),
            r2.astype(jnp.bfloat16))


def kernel(x, enc, dec, codebook):
    bt, tt, _ = x.shape
    n = bt * tt
    blk = 576
    nblk = n // blk

    e = _encoder_x(enc, x).reshape(n, D)

    dec_leaves, dec_tree = _prep_leaves(dec)
    cbt = codebook.T
    cb1, cb2, cb3 = _split3(codebook)
    cb_sq = jnp.sum(codebook ** 2, axis=-1).reshape(1, KCB)

    o2d, idx2d, loss = pl.pallas_call(
        functools.partial(_rvq_dec_kernel, dec_tree, len(dec_leaves), nblk, n),
        grid=(nblk,),
        in_specs=[pl.BlockSpec((blk, D), lambda i: (i, 0)),
                  _const_spec(cbt), _const_spec(cb1), _const_spec(cb2),
                  _const_spec(cb3), _const_spec(cb_sq)]
        + [_const_spec(a) for a in dec_leaves],
        out_specs=[
            pl.BlockSpec((blk, C), lambda i: (i, 0)),
            pl.BlockSpec((blk, NQ), lambda i: (i, 0)),
            pl.BlockSpec((1, NQ), lambda i: (0, 0)),
        ],
        out_shape=[
            jax.ShapeDtypeStruct((n, C), jnp.float32),
            jax.ShapeDtypeStruct((n, NQ), jnp.int32),
            jax.ShapeDtypeStruct((1, NQ), jnp.float32),
        ],
    )(e, cbt, cb1, cb2, cb3, cb_sq, *dec_leaves)

    return (o2d.reshape(bt, tt, C), idx2d.reshape(bt, tt, NQ),
            loss.reshape(NQ))


# speedup vs baseline: 1.0040x; 1.0040x over previous
import functools

import jax
import jax.numpy as jnp
from jax.experimental import pallas as pl

C = 768
D = 64
NQ = 8
KCB = 1024

_HI = jax.lax.Precision.HIGHEST
_DEF = jax.lax.Precision.DEFAULT


def _elu(x):
    return jnp.where(x > 0, x, jnp.exp(jnp.minimum(x, 0.0)) - 1.0)


def _lin_p(p, x, prec):
    del prec
    wh, wl, b = p
    xh = x.astype(jnp.bfloat16)
    xl = (x - xh.astype(jnp.float32)).astype(jnp.bfloat16)
    dn = (((1,), (0,)), ((), ()))
    f = jnp.float32
    y = (jax.lax.dot_general(xl, wh, dn, preferred_element_type=f)
         + jax.lax.dot_general(xh, wl, dn, preferred_element_type=f))
    y = y + jax.lax.dot_general(xh, wh, dn, preferred_element_type=f)
    return y + b


def _ru_p(p, x, prec):
    return x + _lin_p(p[1], _elu(_lin_p(p[0], x, prec)), prec)


def _db_p(p, x, prec):
    x = _elu(_lin_p(p[0], x, prec))
    x = _elu(_ru_p(p[1], x, prec))
    x = _elu(_ru_p(p[2], x, prec))
    return _ru_p(p[3], x, prec)


def _dec_apply(p, x, prec):
    x = _elu(_lin_p(p[0], x, prec))
    x = _elu(_db_p(p[1], x, prec))
    x = _elu(_db_p(p[2], x, prec))
    return _lin_p(p[3], x, prec)


def _lin_x(p, x):
    return x @ p[0] + p[1]


def _ru_x(p, x):
    return x + _lin_x(p[1], jax.nn.elu(_lin_x(p[0], x)))


def _eb_x(p, x):
    x = _ru_x(p[0], x); x = jax.nn.elu(x)
    x = _ru_x(p[1], x); x = jax.nn.elu(x)
    x = _ru_x(p[2], x); x = jax.nn.elu(x)
    return _lin_x(p[3], x)


def _encoder_x(p, x):
    x = _lin_x(p[0], x); x = jax.nn.elu(x)
    x = _eb_x(p[1], x); x = jax.nn.elu(x)
    x = _eb_x(p[2], x); x = jax.nn.elu(x)
    x = _eb_x(p[3], x); x = jax.nn.elu(x)
    x = _eb_x(p[4], x); x = jax.nn.elu(x)
    return _lin_x(p[5], x)


def _rvq_dec_kernel(dec_tree, n_w, nblk, n_tokens,
                    e_ref, *refs):
    cbt_ref = refs[0]
    cbs_refs = refs[1:4]
    cbsq_ref = refs[4]
    w_refs = refs[5:5 + n_w]
    o_ref, idx_ref, loss_ref = refs[5 + n_w:]

    cbt = cbt_ref[...]
    cb_sq = cbsq_ref[...]
    cb1, cb2, cb3 = (r[...] for r in cbs_refs)

    e = e_ref[...]
    bsz = e.shape[0]
    iota = jax.lax.broadcasted_iota(jnp.int32, (bsz, KCB), 1)
    dn = (((1,), (0,)), ((), ()))

    residual = e
    qout = jnp.zeros_like(e)
    idx_cols = []
    loss_cols = []
    for _ in range(NQ):
        rsq = jnp.sum(residual * residual, axis=1, keepdims=True)
        mm = jax.lax.dot_general(residual, cbt, dn, precision=_DEF,
                                 preferred_element_type=jnp.float32)
        d = rsq - 2.0 * mm + cb_sq
        dmin = jnp.min(d, axis=1, keepdims=True)
        idxv = jnp.min(jnp.where(d == dmin, iota, jnp.int32(KCB)),
                       axis=1, keepdims=True)
        oh = (iota == idxv).astype(jnp.bfloat16)
        q12 = (jax.lax.dot_general(oh, cb1, dn, preferred_element_type=jnp.float32)
               + jax.lax.dot_general(oh, cb2, dn, preferred_element_type=jnp.float32))
        quant = q12 + jax.lax.dot_general(oh, cb3, dn,
                                          preferred_element_type=jnp.float32)
        diff = quant - residual
        loss_cols.append(jnp.sum(diff * diff).reshape(1, 1))
        qout = qout + (residual + diff)
        residual = residual - quant
        idx_cols.append(idxv)

    idx_ref[...] = jnp.concatenate(idx_cols, axis=1)

    i = pl.program_id(0)

    @pl.when(i == 0)
    def _zero():
        loss_ref[...] = jnp.zeros_like(loss_ref)

    loss_ref[...] += jnp.concatenate(loss_cols, axis=1)

    @pl.when(i == nblk - 1)
    def _scale():
        loss_ref[...] *= jnp.float32(1.0 / (n_tokens * D))

    dec = jax.tree.unflatten(dec_tree, [r[...] for r in w_refs])
    o_ref[...] = _dec_apply(dec, qout, _HI)


def _split_params(p):
    if (isinstance(p, tuple) and len(p) == 2
            and hasattr(p[0], "ndim") and p[0].ndim == 2
            and hasattr(p[1], "ndim") and p[1].ndim == 1):
        w, b = p
        wh = w.astype(jnp.bfloat16)
        wl = (w - wh.astype(jnp.float32)).astype(jnp.bfloat16)
        return (wh, wl, b.reshape(1, -1))
    return tuple(_split_params(q) for q in p)


def _prep_leaves(params):
    return jax.tree.flatten(_split_params(params))


def _const_spec(a):
    return pl.BlockSpec(a.shape, lambda i: (0,) * a.ndim)


def _split3(v):
    m = jnp.uint32(0xFFFF0000)
    u = jax.lax.bitcast_convert_type(v, jnp.uint32)
    v1f = jax.lax.bitcast_convert_type(u & m, jnp.float32)
    r1 = v - v1f
    u1 = jax.lax.bitcast_convert_type(r1, jnp.uint32)
    v2f = jax.lax.bitcast_convert_type(u1 & m, jnp.float32)
    r2 = r1 - v2f
    return (v1f.astype(jnp.bfloat16), v2f.astype(jnp.bfloat16),
            r2.astype(jnp.bfloat16))


def kernel(x, enc, dec, codebook):
    bt, tt, _ = x.shape
    n = bt * tt
    blk = 576
    nblk = n // blk

    e = _encoder_x(enc, x).reshape(n, D)

    dec_leaves, dec_tree = _prep_leaves(dec)
    cbt = codebook.T
    cb1, cb2, cb3 = _split3(codebook)
    cb_sq = jnp.sum(codebook ** 2, axis=-1).reshape(1, KCB)

    o2d, idx2d, loss = pl.pallas_call(
        functools.partial(_rvq_dec_kernel, dec_tree, len(dec_leaves), nblk, n),
        grid=(nblk,),
        in_specs=[pl.BlockSpec((blk, D), lambda i: (i, 0)),
                  _const_spec(cbt), _const_spec(cb1), _const_spec(cb2),
                  _const_spec(cb3), _const_spec(cb_sq)]
        + [_const_spec(a) for a in dec_leaves],
        out_specs=[
            pl.BlockSpec((blk, C), lambda i: (i, 0)),
            pl.BlockSpec((blk, NQ), lambda i: (i, 0)),
            pl.BlockSpec((1, NQ), lambda i: (0, 0)),
        ],
        out_shape=[
            jax.ShapeDtypeStruct((n, C), jnp.float32),
            jax.ShapeDtypeStruct((n, NQ), jnp.int32),
            jax.ShapeDtypeStruct((1, NQ), jnp.float32),
        ],
    )(e, cbt, cb1, cb2, cb3, cb_sq, *dec_leaves)

    return (o2d.reshape(bt, tt, C), idx2d.reshape(bt, tt, NQ),
            loss.reshape(NQ))


# speedup vs baseline: 1.4189x; 1.4134x over previous
import functools

import jax
import jax.numpy as jnp
from jax.experimental import pallas as pl

C = 768
D = 64
NQ = 8
KCB = 1024

_HI = jax.lax.Precision.HIGHEST
_DEF = jax.lax.Precision.DEFAULT


def _elu(x):
    return jnp.where(x > 0, x, jnp.exp(jnp.minimum(x, 0.0)) - 1.0)


def _lin_p(p, x, prec):
    del prec
    wh, wl, b = p
    xh = x.astype(jnp.bfloat16)
    xl = (x - xh.astype(jnp.float32)).astype(jnp.bfloat16)
    dn = (((1,), (0,)), ((), ()))
    f = jnp.float32
    y = (jax.lax.dot_general(xl, wh, dn, preferred_element_type=f)
         + jax.lax.dot_general(xh, wl, dn, preferred_element_type=f))
    y = y + jax.lax.dot_general(xh, wh, dn, preferred_element_type=f)
    return y + b


def _ru_p(p, x, prec):
    return x + _lin_p(p[1], _elu(_lin_p(p[0], x, prec)), prec)


def _db_p(p, x, prec):
    x = _elu(_lin_p(p[0], x, prec))
    x = _elu(_ru_p(p[1], x, prec))
    x = _elu(_ru_p(p[2], x, prec))
    return _ru_p(p[3], x, prec)


def _dec_apply(p, x, prec):
    x = _elu(_lin_p(p[0], x, prec))
    x = _elu(_db_p(p[1], x, prec))
    x = _elu(_db_p(p[2], x, prec))
    return _lin_p(p[3], x, prec)


def _lin_x(p, x):
    return x @ p[0] + p[1]


def _ru_x(p, x):
    return x + _lin_x(p[1], jax.nn.elu(_lin_x(p[0], x)))


def _eb_x(p, x):
    x = _ru_x(p[0], x); x = jax.nn.elu(x)
    x = _ru_x(p[1], x); x = jax.nn.elu(x)
    x = _ru_x(p[2], x); x = jax.nn.elu(x)
    return _lin_x(p[3], x)


def _encoder_x(p, x):
    x = _lin_x(p[0], x); x = jax.nn.elu(x)
    x = _eb_x(p[1], x); x = jax.nn.elu(x)
    x = _eb_x(p[2], x); x = jax.nn.elu(x)
    x = _eb_x(p[3], x); x = jax.nn.elu(x)
    x = _eb_x(p[4], x); x = jax.nn.elu(x)
    return _lin_x(p[5], x)


def _rvq_dec_kernel(dec_tree, n_w, nblk, n_tokens,
                    e_ref, *refs):
    cbt_ref = refs[0]
    cbcat_ref = refs[1]
    cbsq_ref = refs[2]
    w_refs = refs[3:3 + n_w]
    o_ref, idx_ref, loss_ref = refs[3 + n_w:]

    cbt = cbt_ref[...]
    cb_sq = cbsq_ref[...]
    cbcat = cbcat_ref[...]

    e = e_ref[...]
    bsz = e.shape[0]
    iota = jax.lax.broadcasted_iota(jnp.int32, (bsz, KCB), 1)
    dn = (((1,), (0,)), ((), ()))

    residual = e
    qout = jnp.zeros_like(e)
    idx_cols = []
    loss_cols = []
    for _ in range(NQ):
        rsq = jnp.sum(residual * residual, axis=1, keepdims=True)
        mm = jax.lax.dot_general(residual, cbt, dn, precision=_DEF,
                                 preferred_element_type=jnp.float32)
        d = rsq - 2.0 * mm + cb_sq
        dmin = jnp.min(d, axis=1, keepdims=True)
        idxv = jnp.min(jnp.where(d == dmin, iota, jnp.int32(KCB)),
                       axis=1, keepdims=True)
        oh = (iota == idxv).astype(jnp.bfloat16)
        g = jax.lax.dot_general(oh, cbcat, dn,
                                preferred_element_type=jnp.float32)
        quant = (g[:, :D] + g[:, D:2 * D]) + g[:, 2 * D:]
        diff = quant - residual
        loss_cols.append(jnp.sum(diff * diff).reshape(1, 1))
        qout = qout + (residual + diff)
        residual = residual - quant
        idx_cols.append(idxv)

    idx_ref[...] = jnp.concatenate(idx_cols, axis=1)

    i = pl.program_id(0)

    @pl.when(i == 0)
    def _zero():
        loss_ref[...] = jnp.zeros_like(loss_ref)

    loss_ref[...] += jnp.concatenate(loss_cols, axis=1)

    @pl.when(i == nblk - 1)
    def _scale():
        loss_ref[...] *= jnp.float32(1.0 / (n_tokens * D))

    dec = jax.tree.unflatten(dec_tree, [r[...] for r in w_refs])
    o_ref[...] = _dec_apply(dec, qout, _HI)


def _split_params(p):
    if (isinstance(p, tuple) and len(p) == 2
            and hasattr(p[0], "ndim") and p[0].ndim == 2
            and hasattr(p[1], "ndim") and p[1].ndim == 1):
        w, b = p
        wh = w.astype(jnp.bfloat16)
        wl = (w - wh.astype(jnp.float32)).astype(jnp.bfloat16)
        return (wh, wl, b.reshape(1, -1))
    return tuple(_split_params(q) for q in p)


def _prep_leaves(params):
    return jax.tree.flatten(_split_params(params))


def _const_spec(a):
    return pl.BlockSpec(a.shape, lambda i: (0,) * a.ndim)


def _split3(v):
    m = jnp.uint32(0xFFFF0000)
    u = jax.lax.bitcast_convert_type(v, jnp.uint32)
    v1f = jax.lax.bitcast_convert_type(u & m, jnp.float32)
    r1 = v - v1f
    u1 = jax.lax.bitcast_convert_type(r1, jnp.uint32)
    v2f = jax.lax.bitcast_convert_type(u1 & m, jnp.float32)
    r2 = r1 - v2f
    return (v1f.astype(jnp.bfloat16), v2f.astype(jnp.bfloat16),
            r2.astype(jnp.bfloat16))


def kernel(x, enc, dec, codebook):
    bt, tt, _ = x.shape
    n = bt * tt
    blk = 1152
    nblk = n // blk

    e = _encoder_x(enc, x).reshape(n, D)

    dec_leaves, dec_tree = _prep_leaves(dec)
    cbt = codebook.T
    cbcat = jnp.concatenate(_split3(codebook), axis=1)
    cb_sq = jnp.sum(codebook ** 2, axis=-1).reshape(1, KCB)

    o2d, idx2d, loss = pl.pallas_call(
        functools.partial(_rvq_dec_kernel, dec_tree, len(dec_leaves), nblk, n),
        grid=(nblk,),
        in_specs=[pl.BlockSpec((blk, D), lambda i: (i, 0)),
                  _const_spec(cbt), _const_spec(cbcat), _const_spec(cb_sq)]
        + [_const_spec(a) for a in dec_leaves],
        out_specs=[
            pl.BlockSpec((blk, C), lambda i: (i, 0)),
            pl.BlockSpec((blk, NQ), lambda i: (i, 0)),
            pl.BlockSpec((1, NQ), lambda i: (0, 0)),
        ],
        out_shape=[
            jax.ShapeDtypeStruct((n, C), jnp.float32),
            jax.ShapeDtypeStruct((n, NQ), jnp.int32),
            jax.ShapeDtypeStruct((1, NQ), jnp.float32),
        ],
    )(e, cbt, cbcat, cb_sq, *dec_leaves)

    return (o2d.reshape(bt, tt, C), idx2d.reshape(bt, tt, NQ),
            loss.reshape(NQ))
